# baseline (speedup 1.0000x reference)
import jax
import jax.numpy as jnp
from jax.experimental import pallas as pl

_N = 26
_JB = 40


def _onehot_body(idxt_ref, out_ref):
    x = idxt_ref[...]
    k = pl.program_id(1)
    out_ref[...] = jnp.where(x[None, :, :] == k, 1, 0).astype(jnp.int32)


def kernel(idxs_vec):
    b, l = idxs_vec.shape
    idxt = idxs_vec.T
    out3 = pl.pallas_call(
        _onehot_body,
        grid=(l // _JB, _N),
        in_specs=[pl.BlockSpec((_JB, b), lambda j, k: (j, 0))],
        out_specs=pl.BlockSpec((1, _JB, b), lambda j, k: (k, j, 0)),
        out_shape=jax.ShapeDtypeStruct((_N, l, b), jnp.int32),
    )(idxt)
    return jnp.transpose(out3, (2, 1, 0))
